# baseline (speedup 1.0000x reference)
import jax
import jax.numpy as jnp
from jax.experimental import pallas as pl


def _gat_conv(x, src, dst, W, a_src, a_dst, b):
    n = x.shape[0]
    loop = jnp.arange(n, dtype=src.dtype)
    s = jnp.concatenate([src, loop])
    d = jnp.concatenate([dst, loop])
    h = x @ W
    al_s = h @ a_src
    al_d = h @ a_dst
    e = jax.nn.leaky_relu(al_s[s] + al_d[d], 0.2)
    m = jax.ops.segment_max(e, d, num_segments=n)
    m = jnp.where(jnp.isfinite(m), m, 0.0)
    ex = jnp.exp(e - m[d])
    denom = jax.ops.segment_sum(ex, d, num_segments=n)
    alpha = ex / (denom[d] + 1e-16)
    out = jax.ops.segment_sum(h[s] * alpha[:, None], d, num_segments=n)
    return out + b


def _branch(x, edge_index, p1, p2):
    src, dst = edge_index[0], edge_index[1]
    h = jax.nn.relu(_gat_conv(x, src, dst, *p1))
    h = jax.nn.relu(_gat_conv(h, src, dst, *p2))
    return h


def _scatter_mean(x, seg, num_segments):
    s = jax.ops.segment_sum(x, seg, num_segments=num_segments)
    cnt = jax.ops.segment_sum(jnp.ones((x.shape[0],), x.dtype), seg, num_segments=num_segments)
    return s / jnp.maximum(cnt, 1.0)[:, None]


def _fc_head_kernel(z_ref, w_ref, b_ref, o_ref):
    out = z_ref[...] @ w_ref[...] + b_ref[...]
    out = out - jax.scipy.special.logsumexp(out, axis=1, keepdims=True)
    o_ref[...] = out


def kernel(x, TD_edge_index, BU_edge_index, batch,
           W_td1, asrc_td1, adst_td1, b_td1,
           W_td2, asrc_td2, adst_td2, b_td2,
           W_bu1, asrc_bu1, adst_bu1, b_bu1,
           W_bu2, asrc_bu2, adst_bu2, b_bu2,
           fc_W, fc_b):
    td = _branch(x, TD_edge_index,
                 (W_td1, asrc_td1, adst_td1, b_td1),
                 (W_td2, asrc_td2, adst_td2, b_td2))
    bu = _branch(x, BU_edge_index,
                 (W_bu1, asrc_bu1, adst_bu1, b_bu1),
                 (W_bu2, asrc_bu2, adst_bu2, b_bu2))
    td_g = _scatter_mean(td, batch, 128)
    bu_g = _scatter_mean(bu, batch, 128)
    z = jnp.concatenate([bu_g, td_g], axis=1)
    out = pl.pallas_call(
        _fc_head_kernel,
        out_shape=jax.ShapeDtypeStruct((z.shape[0], fc_W.shape[1]), z.dtype),
    )(z, fc_W, fc_b[None, :])
    return out


# speedup vs baseline: 22.8545x; 22.8545x over previous
import dataclasses
import functools

import jax
import jax.numpy as jnp
from jax import lax
from jax.experimental import pallas as pl
from jax.experimental.pallas import tpu as pltpu
from jax.experimental.pallas import tpu_sc as plsc

N = 10000
NPAD = 10112
D = 128
E = 320000
E2 = E + N
CHUNK = 128
NSUB = 16
NCORE = 2
KCH = -(-E2 // (CHUNK * NSUB * NCORE))
E2PAD = KCH * CHUNK * NSUB * NCORE
STRIPE = NPAD // NSUB

_mesh = plsc.VectorSubcoreMesh(core_axis_name="c", subcore_axis_name="s")

_sc_params = pltpu.CompilerParams()
if "needs_layout_passes" in pltpu.CompilerParams.__dataclass_fields__:
    _sc_params = dataclasses.replace(_sc_params, needs_layout_passes=False)


@functools.partial(
    pl.kernel,
    out_type=(
        jax.ShapeDtypeStruct((NCORE, NPAD, D), jnp.float32),
        jax.ShapeDtypeStruct((NCORE, NSUB, NPAD), jnp.float32),
    ),
    mesh=_mesh,
    scratch_types=[
        pltpu.VMEM((2, CHUNK), jnp.int32),
        pltpu.VMEM((CHUNK, D), jnp.float32),
        pltpu.VMEM((CHUNK,), jnp.float32),
        pltpu.VMEM((NPAD,), jnp.float32),
        pltpu.VMEM((NPAD,), jnp.float32),
        pltpu.VMEM((NPAD,), jnp.float32),
        pltpu.VMEM((16,), jnp.float32),
        pltpu.VMEM_SHARED((NPAD, D), jnp.float32),
        pltpu.SemaphoreType.DMA,
    ],
    compiler_params=_sc_params,
)
def _sc_gat_aggregate(h_hbm, s_hbm, d_hbm, als_hbm, ald_hbm, c_hbm, z_hbm,
                      out_hbm, den_hbm, idx_v, rows_v, ex_v, als_v, ald_v,
                      den_v, c_v, acc_sh, sem):
    cid = lax.axis_index("c")
    sid = lax.axis_index("s")
    wid = sid * NCORE + cid

    pltpu.sync_copy(als_hbm, als_v)
    pltpu.sync_copy(ald_hbm, ald_v)
    pltpu.sync_copy(c_hbm, c_v)
    pltpu.sync_copy(z_hbm.at[pl.ds(sid * STRIPE, STRIPE)],
                    acc_sh.at[pl.ds(sid * STRIPE, STRIPE)])

    zeros16 = jnp.zeros((16,), jnp.float32)

    @pl.loop(0, NPAD, step=16)
    def _zden(i):
        den_v[pl.ds(i, 16)] = zeros16

    plsc.subcore_barrier()

    c_reg = c_v[...]
    lane_iota = lax.iota(jnp.int32, 16)

    @pl.loop(0, KCH)
    def _chunk(k):
        base = (wid * KCH + k) * CHUNK
        pltpu.sync_copy(s_hbm.at[pl.ds(base, CHUNK)], idx_v.at[0])
        pltpu.sync_copy(d_hbm.at[pl.ds(base, CHUNK)], idx_v.at[1])
        gat = pltpu.async_copy(h_hbm.at[idx_v.at[0]], rows_v, sem)

        @pl.loop(0, CHUNK, step=16)
        def _ex(i):
            si = idx_v[0, pl.ds(i, 16)]
            di = idx_v[1, pl.ds(i, 16)]
            u = plsc.load_gather(als_v, [si]) + plsc.load_gather(ald_v, [di])
            e = jnp.maximum(u, u * 0.2)
            exv = jnp.exp(e - c_reg)
            ex_v[pl.ds(i, 16)] = exv
            for lane in range(16):
                plsc.addupdate_scatter(den_v, [di], exv,
                                       mask=lane_iota == lane)

        gat.wait()

        @pl.loop(0, CHUNK)
        def _scale(e):
            bex = plsc.load_gather(ex_v, [jnp.full((16,), e, jnp.int32)])
            for j in range(D // 16):
                rows_v[e, pl.ds(j * 16, 16)] = (
                    rows_v[e, pl.ds(j * 16, 16)] * bex)

        pltpu.sync_copy(rows_v, acc_sh.at[idx_v.at[1]], add=True)

    plsc.subcore_barrier()
    pltpu.sync_copy(acc_sh.at[pl.ds(sid * STRIPE, STRIPE)],
                    out_hbm.at[cid, pl.ds(sid * STRIPE, STRIPE)])
    pltpu.sync_copy(den_v, den_hbm.at[cid, sid])


def _compute_h_al(xin, w_ref, asrc_ref, adst_ref, h_ref, als_ref, ald_ref,
                  c_ref):
    h = xin @ w_ref[...]
    h_ref[...] = h
    als = h @ asrc_ref[...]
    ald = h @ adst_ref[...]
    als_ref[...] = als
    ald_ref[...] = ald
    u = jnp.max(als) + jnp.max(ald)
    c = jnp.maximum(u, u * 0.2)
    c_ref[...] = jnp.full((1, 128), c, jnp.float32)


def _pre_body(x_ref, w_ref, asrc_ref, adst_ref, h_ref, als_ref, ald_ref,
              c_ref):
    _compute_h_al(x_ref[...], w_ref, asrc_ref, adst_ref, h_ref, als_ref,
                  ald_ref, c_ref)


def _hidden_from_acc(acc_ref, den_ref, b_ref):
    acc = acc_ref[0] + acc_ref[1]
    den = jnp.sum(den_ref[...], axis=(0, 1)).reshape(NPAD, 1)
    hid = jnp.maximum(acc / jnp.maximum(den, 1e-30) + b_ref[...], 0.0)
    rows = lax.broadcasted_iota(jnp.int32, (NPAD, 1), 0)
    return jnp.where(rows < N, hid, 0.0)


def _mid_body(acc_ref, den_ref, b_ref, w_ref, asrc_ref, adst_ref, h_ref,
              als_ref, ald_ref, c_ref):
    hid = _hidden_from_acc(acc_ref, den_ref, b_ref)
    _compute_h_al(hid, w_ref, asrc_ref, adst_ref, h_ref, als_ref, ald_ref,
                  c_ref)


def _post2_body(acc_ref, den_ref, b_ref, batch_ref, pooled_ref):
    hid = _hidden_from_acc(acc_ref, den_ref, b_ref)
    gids = lax.broadcasted_iota(jnp.int32, (128, 1), 0)
    oh = (batch_ref[...] == gids).astype(jnp.float32)
    sums = oh @ hid
    cnt = jnp.sum(oh, axis=1, keepdims=True)
    pooled_ref[...] = sums / jnp.maximum(cnt, 1.0)


def _head_body(bug_ref, tdg_ref, w_ref, b_ref, o_ref):
    z = jnp.concatenate([bug_ref[...], tdg_ref[...]], axis=1)
    o = z @ w_ref[...] + b_ref[...]
    m = jnp.max(o, axis=1, keepdims=True)
    o_ref[...] = o - m - jnp.log(jnp.sum(jnp.exp(o - m), axis=1,
                                         keepdims=True))


_f32 = jnp.float32

_tc_out4 = (
    jax.ShapeDtypeStruct((NPAD, D), _f32),
    jax.ShapeDtypeStruct((NPAD, 1), _f32),
    jax.ShapeDtypeStruct((NPAD, 1), _f32),
    jax.ShapeDtypeStruct((1, 128), _f32),
)


def _tc_pre(x_pad, W, a_src, a_dst):
    return pl.pallas_call(_pre_body, out_shape=_tc_out4)(
        x_pad, W, a_src.reshape(D, 1), a_dst.reshape(D, 1))


def _tc_mid(acc, den, b, W, a_src, a_dst):
    return pl.pallas_call(_mid_body, out_shape=_tc_out4)(
        acc, den, b.reshape(1, D), W, a_src.reshape(D, 1),
        a_dst.reshape(D, 1))


def _tc_post2(acc, den, b, batch_pad):
    return pl.pallas_call(
        _post2_body,
        out_shape=jax.ShapeDtypeStruct((128, D), _f32),
    )(acc, den, b.reshape(1, D), batch_pad.reshape(1, NPAD))


def _tc_head(bu_g, td_g, fc_W, fc_b):
    return pl.pallas_call(
        _head_body,
        out_shape=jax.ShapeDtypeStruct((128, fc_W.shape[1]), _f32),
    )(bu_g, td_g, fc_W, fc_b.reshape(1, fc_W.shape[1]))


def _sc_conv(h, als, ald, c, s_idx, d_idx, zeros_acc):
    return _sc_gat_aggregate(h, s_idx, d_idx, als.reshape(NPAD),
                             ald.reshape(NPAD), c[0, :16], zeros_acc)


def kernel(x, TD_edge_index, BU_edge_index, batch,
           W_td1, asrc_td1, adst_td1, b_td1,
           W_td2, asrc_td2, adst_td2, b_td2,
           W_bu1, asrc_bu1, adst_bu1, b_bu1,
           W_bu2, asrc_bu2, adst_bu2, b_bu2,
           fc_W, fc_b):
    x_pad = jnp.pad(x, ((0, NPAD - N), (0, 0)))
    loop = jnp.arange(N, dtype=jnp.int32)
    padi = jnp.full((E2PAD - E2,), N, jnp.int32)
    s_td = jnp.concatenate([TD_edge_index[0], loop, padi])
    d_td = jnp.concatenate([TD_edge_index[1], loop, padi])
    s_bu = jnp.concatenate([BU_edge_index[0], loop, padi])
    d_bu = jnp.concatenate([BU_edge_index[1], loop, padi])
    batch_pad = jnp.pad(batch, (0, NPAD - N), constant_values=1 << 27)
    zeros_acc = jnp.zeros((NPAD, D), _f32)

    def branch(s_idx, d_idx, p1, p2):
        (W1, a_s1, a_d1, b1) = p1
        (W2, a_s2, a_d2, b2) = p2
        h1, als1, ald1, c1 = _tc_pre(x_pad, W1, a_s1, a_d1)
        acc1, den1 = _sc_conv(h1, als1, ald1, c1, s_idx, d_idx, zeros_acc)
        h2, als2, ald2, c2 = _tc_mid(acc1, den1, b1, W2, a_s2, a_d2)
        acc2, den2 = _sc_conv(h2, als2, ald2, c2, s_idx, d_idx, zeros_acc)
        return _tc_post2(acc2, den2, b2, batch_pad)

    td_g = branch(s_td, d_td,
                  (W_td1, asrc_td1, adst_td1, b_td1),
                  (W_td2, asrc_td2, adst_td2, b_td2))
    bu_g = branch(s_bu, d_bu,
                  (W_bu1, asrc_bu1, adst_bu1, b_bu1),
                  (W_bu2, asrc_bu2, adst_bu2, b_bu2))
    return _tc_head(bu_g, td_g, fc_W, fc_b)


# speedup vs baseline: 29.0135x; 1.2695x over previous
import dataclasses
import functools

import jax
import jax.numpy as jnp
from jax import lax
from jax.experimental import pallas as pl
from jax.experimental.pallas import tpu as pltpu
from jax.experimental.pallas import tpu_sc as plsc

N = 10000
NPAD = 10112
D = 128
E = 320000
E2 = E + N
CHUNK = 64
NSUB = 16
NCORE = 2
KCH = -(-E2 // (CHUNK * NSUB * NCORE))
E2PAD = KCH * CHUNK * NSUB * NCORE
NCHT = E2PAD // CHUNK
STRIPE = NPAD // NSUB

_mesh = plsc.VectorSubcoreMesh(core_axis_name="c", subcore_axis_name="s")

_sc_params = pltpu.CompilerParams()
if "needs_layout_passes" in pltpu.CompilerParams.__dataclass_fields__:
    _sc_params = dataclasses.replace(_sc_params, needs_layout_passes=False)


@functools.partial(
    pl.kernel,
    out_type=(
        jax.ShapeDtypeStruct((NCORE, NPAD, D), jnp.float32),
        jax.ShapeDtypeStruct((NCORE, NSUB, NPAD), jnp.float32),
    ),
    mesh=_mesh,
    scratch_types=[
        pltpu.VMEM((2, 2, CHUNK), jnp.int32),
        pltpu.VMEM((2, CHUNK, D), jnp.float32),
        pltpu.VMEM((2, CHUNK), jnp.float32),
        pltpu.VMEM((NPAD,), jnp.float32),
        pltpu.VMEM((NPAD,), jnp.float32),
        pltpu.VMEM((NPAD,), jnp.float32),
        pltpu.VMEM((16,), jnp.float32),
        pltpu.VMEM_SHARED((NPAD, D), jnp.float32),
        pltpu.SemaphoreType.DMA,
        pltpu.SemaphoreType.DMA,
    ],
    compiler_params=_sc_params,
)
def _sc_gat_aggregate(h_hbm, sd_hbm, als_hbm, ald_hbm, c_hbm, z_hbm,
                      out_hbm, den_hbm, idx_v, rows_v, ex_v, als_v, ald_v,
                      den_v, c_v, acc_sh, semg0, semg1):
    cid = lax.axis_index("c")
    sid = lax.axis_index("s")
    wid = sid * NCORE + cid
    first = wid * KCH
    semg = (semg0, semg1)

    pltpu.sync_copy(als_hbm, als_v)
    pltpu.sync_copy(ald_hbm, ald_v)
    pltpu.sync_copy(c_hbm, c_v)
    pltpu.sync_copy(z_hbm.at[pl.ds(sid * STRIPE, STRIPE)],
                    acc_sh.at[pl.ds(sid * STRIPE, STRIPE)])

    zeros16 = jnp.zeros((16,), jnp.float32)

    @pl.loop(0, NPAD, step=16)
    def _zden(i):
        den_v[pl.ds(i, 16)] = zeros16

    plsc.subcore_barrier()

    c_reg = c_v[...]
    lane_iota = lax.iota(jnp.int32, 16)

    for b in range(2):
        pltpu.sync_copy(sd_hbm.at[first + b], idx_v.at[b])
        pltpu.async_copy(h_hbm.at[idx_v.at[b, 0]], rows_v.at[b], semg[b])

    @pl.loop(0, KCH, step=2)
    def _chunk(k):
        for b in range(2):
            ck = k + b
            pltpu.make_async_copy(h_hbm.at[idx_v.at[b, 0]], rows_v.at[b],
                                  semg[b]).wait()

            @pl.loop(0, CHUNK, step=16)
            def _ex(i):
                si = idx_v[b, 0, pl.ds(i, 16)]
                di = idx_v[b, 1, pl.ds(i, 16)]
                u = (plsc.load_gather(als_v, [si])
                     + plsc.load_gather(ald_v, [di]))
                e = jnp.maximum(u, u * 0.2)
                exv = jnp.exp(e - c_reg)
                ex_v[b, pl.ds(i, 16)] = exv
                for lane in range(16):
                    plsc.addupdate_scatter(den_v, [di], exv,
                                           mask=lane_iota == lane)

            @pl.loop(0, CHUNK)
            def _scale(e):
                bex = plsc.load_gather(ex_v.at[b],
                                       [jnp.full((16,), e, jnp.int32)])
                for j in range(D // 16):
                    rows_v[b, e, pl.ds(j * 16, 16)] = (
                        rows_v[b, e, pl.ds(j * 16, 16)] * bex)

            pltpu.sync_copy(rows_v.at[b], acc_sh.at[idx_v.at[b, 1]],
                            add=True)

            @pl.when(ck + 2 < KCH)
            def _issue_next():
                pltpu.sync_copy(sd_hbm.at[first + ck + 2], idx_v.at[b])
                pltpu.async_copy(h_hbm.at[idx_v.at[b, 0]], rows_v.at[b],
                                 semg[b])

    plsc.subcore_barrier()
    pltpu.sync_copy(acc_sh.at[pl.ds(sid * STRIPE, STRIPE)],
                    out_hbm.at[cid, pl.ds(sid * STRIPE, STRIPE)])
    pltpu.sync_copy(den_v, den_hbm.at[cid, sid])


def _compute_h_al(xin, w_ref, asrc_ref, adst_ref, h_ref, als_ref, ald_ref,
                  c_ref):
    h = xin @ w_ref[...]
    h_ref[...] = h
    als = h @ asrc_ref[...]
    ald = h @ adst_ref[...]
    als_ref[...] = als
    ald_ref[...] = ald
    u = jnp.max(als) + jnp.max(ald)
    c = jnp.maximum(u, u * 0.2)
    c_ref[...] = jnp.full((1, 128), c, jnp.float32)


def _pre_body(x_ref, w_ref, asrc_ref, adst_ref, h_ref, als_ref, ald_ref,
              c_ref):
    _compute_h_al(x_ref[...], w_ref, asrc_ref, adst_ref, h_ref, als_ref,
                  ald_ref, c_ref)


def _hidden_from_acc(acc_ref, den_ref, b_ref):
    acc = acc_ref[0] + acc_ref[1]
    den = jnp.sum(den_ref[...], axis=(0, 1)).reshape(NPAD, 1)
    hid = jnp.maximum(acc / jnp.maximum(den, 1e-30) + b_ref[...], 0.0)
    rows = lax.broadcasted_iota(jnp.int32, (NPAD, 1), 0)
    return jnp.where(rows < N, hid, 0.0)


def _mid_body(acc_ref, den_ref, b_ref, w_ref, asrc_ref, adst_ref, h_ref,
              als_ref, ald_ref, c_ref):
    hid = _hidden_from_acc(acc_ref, den_ref, b_ref)
    _compute_h_al(hid, w_ref, asrc_ref, adst_ref, h_ref, als_ref, ald_ref,
                  c_ref)


def _post2_body(acc_ref, den_ref, b_ref, batch_ref, pooled_ref):
    hid = _hidden_from_acc(acc_ref, den_ref, b_ref)
    gids = lax.broadcasted_iota(jnp.int32, (128, 1), 0)
    oh = (batch_ref[...] == gids).astype(jnp.float32)
    sums = oh @ hid
    cnt = jnp.sum(oh, axis=1, keepdims=True)
    pooled_ref[...] = sums / jnp.maximum(cnt, 1.0)


def _head_body(bug_ref, tdg_ref, w_ref, b_ref, o_ref):
    z = jnp.concatenate([bug_ref[...], tdg_ref[...]], axis=1)
    o = z @ w_ref[...] + b_ref[...]
    m = jnp.max(o, axis=1, keepdims=True)
    o_ref[...] = o - m - jnp.log(jnp.sum(jnp.exp(o - m), axis=1,
                                         keepdims=True))


_f32 = jnp.float32

_tc_out4 = (
    jax.ShapeDtypeStruct((NPAD, D), _f32),
    jax.ShapeDtypeStruct((NPAD, 1), _f32),
    jax.ShapeDtypeStruct((NPAD, 1), _f32),
    jax.ShapeDtypeStruct((1, 128), _f32),
)


def _tc_pre(x_pad, W, a_src, a_dst):
    return pl.pallas_call(_pre_body, out_shape=_tc_out4)(
        x_pad, W, a_src.reshape(D, 1), a_dst.reshape(D, 1))


def _tc_mid(acc, den, b, W, a_src, a_dst):
    return pl.pallas_call(_mid_body, out_shape=_tc_out4)(
        acc, den, b.reshape(1, D), W, a_src.reshape(D, 1),
        a_dst.reshape(D, 1))


def _tc_post2(acc, den, b, batch_pad):
    return pl.pallas_call(
        _post2_body,
        out_shape=jax.ShapeDtypeStruct((128, D), _f32),
    )(acc, den, b.reshape(1, D), batch_pad.reshape(1, NPAD))


def _tc_head(bu_g, td_g, fc_W, fc_b):
    return pl.pallas_call(
        _head_body,
        out_shape=jax.ShapeDtypeStruct((128, fc_W.shape[1]), _f32),
    )(bu_g, td_g, fc_W, fc_b.reshape(1, fc_W.shape[1]))


def _sc_conv(h, als, ald, c, sd_idx, zeros_acc):
    return _sc_gat_aggregate(h, sd_idx, als.reshape(NPAD),
                             ald.reshape(NPAD), c[0, :16], zeros_acc)


def kernel(x, TD_edge_index, BU_edge_index, batch,
           W_td1, asrc_td1, adst_td1, b_td1,
           W_td2, asrc_td2, adst_td2, b_td2,
           W_bu1, asrc_bu1, adst_bu1, b_bu1,
           W_bu2, asrc_bu2, adst_bu2, b_bu2,
           fc_W, fc_b):
    x_pad = jnp.pad(x, ((0, NPAD - N), (0, 0)))
    loop = jnp.arange(N, dtype=jnp.int32)
    padi = jnp.full((E2PAD - E2,), N, jnp.int32)

    def pack_sd(ei):
        s = jnp.concatenate([ei[0], loop, padi]).reshape(NCHT, 1, CHUNK)
        d = jnp.concatenate([ei[1], loop, padi]).reshape(NCHT, 1, CHUNK)
        return jnp.concatenate([s, d], axis=1)

    sd_td = pack_sd(TD_edge_index)
    sd_bu = pack_sd(BU_edge_index)
    batch_pad = jnp.pad(batch, (0, NPAD - N), constant_values=1 << 27)
    zeros_acc = jnp.zeros((NPAD, D), _f32)

    def branch(sd_idx, p1, p2):
        (W1, a_s1, a_d1, b1) = p1
        (W2, a_s2, a_d2, b2) = p2
        h1, als1, ald1, c1 = _tc_pre(x_pad, W1, a_s1, a_d1)
        acc1, den1 = _sc_conv(h1, als1, ald1, c1, sd_idx, zeros_acc)
        h2, als2, ald2, c2 = _tc_mid(acc1, den1, b1, W2, a_s2, a_d2)
        acc2, den2 = _sc_conv(h2, als2, ald2, c2, sd_idx, zeros_acc)
        return _tc_post2(acc2, den2, b2, batch_pad)

    td_g = branch(sd_td,
                  (W_td1, asrc_td1, adst_td1, b_td1),
                  (W_td2, asrc_td2, adst_td2, b_td2))
    bu_g = branch(sd_bu,
                  (W_bu1, asrc_bu1, adst_bu1, b_bu1),
                  (W_bu2, asrc_bu2, adst_bu2, b_bu2))
    return _tc_head(bu_g, td_g, fc_W, fc_b)
